# baseline (speedup 1.0000x reference)
import functools

import jax
import jax.numpy as jnp
from jax import lax
from jax.experimental import pallas as pl
from jax.experimental.pallas import tpu as pltpu
from jax.experimental.pallas import tpu_sc as plsc

N = 10000
E = 320000
D_IN = 128
H = 64
D_OUT = 128

NC = 2
NS = 16
NW = NC * NS

B = 128
NK = 80
NBUF = 2
E_PAD = NW * NK * B
STRIPE = 640
NP = NS * STRIPE
DUMMY = N

_f32 = jnp.float32


def _sc_body(y_hbm, src_hbm, dst_hbm, part0, part1, src_v, dst_v, rows, tbl,
             acc, gs0, gs1, gs2, ss0):
    c = lax.axis_index("c")
    s = lax.axis_index("s")
    wid = c * NS + s
    base = s * STRIPE

    d_src = pltpu.async_copy(src_hbm.at[wid], src_v, gs1)
    d_dst = pltpu.async_copy(dst_hbm.at[wid], dst_v, gs2)
    d_tbl = pltpu.async_copy(
        y_hbm.at[pl.ds(base, STRIPE)], tbl.at[pl.ds(base, STRIPE)], gs0)

    z = jnp.zeros((16,), _f32)

    def zrow(r, carry):
        for g in range(H // 16):
            rows[0, r, pl.ds(g * 16, 16)] = z
        return carry

    lax.fori_loop(0, B, zrow, 0)
    d_z = [
        pltpu.async_copy(rows.at[0], acc.at[pl.ds(base + j * B, B)], ss0)
        for j in range(STRIPE // B)
    ]
    for d in (d_src, d_dst, d_tbl, *d_z):
        d.wait()
    plsc.subcore_barrier()

    pltpu.async_copy(tbl.at[src_v.at[0]], rows.at[0], gs0)

    def chunk(k, carry):
        b = lax.rem(k, 2)
        nxt = jnp.minimum(k + 1, NK - 1)
        pltpu.make_async_copy(tbl.at[src_v.at[k]], rows.at[b], gs0).wait()
        pltpu.async_copy(tbl.at[src_v.at[nxt]], rows.at[1 - b], gs0)
        pltpu.sync_copy(rows.at[b], acc.at[dst_v.at[k]], add=True)
        return carry

    lax.fori_loop(0, NK, chunk, 0)
    pltpu.make_async_copy(tbl.at[src_v.at[0]], rows.at[NK % 2], gs0).wait()
    plsc.subcore_barrier()

    @pl.when(c == 0)
    def _():
        pltpu.sync_copy(acc.at[pl.ds(base, STRIPE)], part0.at[pl.ds(base, STRIPE)])

    @pl.when(c == 1)
    def _():
        pltpu.sync_copy(acc.at[pl.ds(base, STRIPE)], part1.at[pl.ds(base, STRIPE)])


_sc_segsum = functools.partial(
    pl.kernel,
    out_type=(
        jax.ShapeDtypeStruct((NP, H), _f32),
        jax.ShapeDtypeStruct((NP, H), _f32),
    ),
    mesh=plsc.VectorSubcoreMesh(core_axis_name="c", subcore_axis_name="s"),
    compiler_params=pltpu.CompilerParams(use_tc_tiling_on_sc=False),
    scratch_types=[
        pltpu.VMEM((NK, B), jnp.int32),
        pltpu.VMEM((NK, B), jnp.int32),
        pltpu.VMEM((NBUF, B, H), _f32),
        pltpu.VMEM_SHARED((NP, H), _f32),
        pltpu.VMEM_SHARED((NP, H), _f32),
    ] + [pltpu.SemaphoreType.DMA] * 4,
)(_sc_body)


_DN = (((1,), (1,)), ((), ()))


def _lrelu(v):
    return jnp.where(v >= 0, v, 0.01 * v)


def _t0_body(x_ref, wrel_ref, wroot_ref, b_ref, y_ref, r_ref):
    y_ref[...] = lax.dot_general(x_ref[...], wrel_ref[...], _DN,
                                 preferred_element_type=_f32)
    r_ref[...] = b_ref[...] + lax.dot_general(x_ref[...], wroot_ref[...], _DN,
                                              preferred_element_type=_f32)


def _comb1_body(p0_ref, p1_ref, r_ref, wrel_ref, wroot_ref, b_ref, y_ref,
                r2_ref):
    h = _lrelu(p0_ref[...] + p1_ref[...] + r_ref[...])
    y_ref[...] = lax.dot_general(h, wrel_ref[...], _DN,
                                 preferred_element_type=_f32)
    r2_ref[...] = b_ref[...] + lax.dot_general(h, wroot_ref[...], _DN,
                                               preferred_element_type=_f32)


def _comb2_body(p0_ref, p1_ref, r_ref, wroot_ref, b_ref, h_ref, r2_ref):
    h = _lrelu(p0_ref[...] + p1_ref[...] + r_ref[...])
    h_ref[...] = h
    r2_ref[...] = b_ref[...] + lax.dot_general(h, wroot_ref[...], _DN,
                                               preferred_element_type=_f32)


def _comb3_body(p0_ref, p1_ref, wrel_ref, r_ref, out_ref):
    agg = p0_ref[...] + p1_ref[...]
    rel = lax.dot_general(agg, wrel_ref[...], _DN,
                          preferred_element_type=_f32)
    out_ref[...] = _lrelu(rel + r_ref[...])


_GRID = (NP // 1024,)


def _row_spec(w):
    return pl.BlockSpec((1024, w), lambda i: (i, 0))


def _full_spec(shape):
    return pl.BlockSpec(shape, lambda i: (0,) * len(shape))


def _tc_t0(x, wrel, wroot, b):
    return pl.pallas_call(
        _t0_body,
        grid=_GRID,
        in_specs=[_row_spec(D_IN), _full_spec(wrel.shape),
                  _full_spec(wroot.shape), _full_spec((1, H))],
        out_specs=[_row_spec(H), _row_spec(H)],
        out_shape=[jax.ShapeDtypeStruct((NP, H), _f32),
                   jax.ShapeDtypeStruct((NP, H), _f32)],
    )(x, wrel, wroot, b.reshape(1, H))


def _tc_comb1(p0, p1, r, wrel, wroot, b):
    return pl.pallas_call(
        _comb1_body,
        grid=_GRID,
        in_specs=[_row_spec(H), _row_spec(H), _row_spec(H),
                  _full_spec(wrel.shape), _full_spec(wroot.shape),
                  _full_spec((1, H))],
        out_specs=[_row_spec(H), _row_spec(H)],
        out_shape=[jax.ShapeDtypeStruct((NP, H), _f32),
                   jax.ShapeDtypeStruct((NP, H), _f32)],
    )(p0, p1, r, wrel, wroot, b.reshape(1, H))


def _tc_comb2(p0, p1, r, wroot, b):
    return pl.pallas_call(
        _comb2_body,
        grid=_GRID,
        in_specs=[_row_spec(H), _row_spec(H), _row_spec(H),
                  _full_spec(wroot.shape), _full_spec((1, D_OUT))],
        out_specs=[_row_spec(H), _row_spec(D_OUT)],
        out_shape=[jax.ShapeDtypeStruct((NP, H), _f32),
                   jax.ShapeDtypeStruct((NP, D_OUT), _f32)],
    )(p0, p1, r, wroot, b.reshape(1, D_OUT))


def _nrow_spec(w):
    return pl.BlockSpec((1000, w), lambda i: (i, 0))


def _tc_comb3(p0, p1, wrel, r):
    return pl.pallas_call(
        _comb3_body,
        grid=(N // 1000,),
        in_specs=[_nrow_spec(H), _nrow_spec(H), _full_spec(wrel.shape),
                  _nrow_spec(D_OUT)],
        out_specs=_nrow_spec(D_OUT),
        out_shape=jax.ShapeDtypeStruct((N, D_OUT), _f32),
    )(p0, p1, wrel, r)


def kernel(x, edge_index, batch, Wrel0, brel0, Wroot0, Wrel1, brel1, Wroot1,
           Wrel2, brel2, Wroot2):
    pad = E_PAD - E
    src = jnp.concatenate([edge_index[0], jnp.zeros((pad,), jnp.int32)])
    dst = jnp.concatenate([edge_index[1], jnp.full((pad,), DUMMY, jnp.int32)])
    src3 = src.reshape(NW, NK, B)
    dst3 = dst.reshape(NW, NK, B)

    x_p = jnp.pad(x, ((0, NP - N), (0, 0)))

    y0, r0 = _tc_t0(x_p, Wrel0, Wroot0, brel0)
    a0, b0 = _sc_segsum(y0, src3, dst3)
    y1, r1 = _tc_comb1(a0, b0, r0, Wrel1, Wroot1, brel1)
    a1, b1 = _sc_segsum(y1, src3, dst3)
    h2, r2 = _tc_comb2(a1, b1, r1, Wroot2, brel2)
    a2, b2 = _sc_segsum(h2, src3, dst3)
    return _tc_comb3(a2, b2, Wrel2, r2)


# speedup vs baseline: 1.0035x; 1.0035x over previous
import functools

import jax
import jax.numpy as jnp
from jax import lax
from jax.experimental import pallas as pl
from jax.experimental.pallas import tpu as pltpu
from jax.experimental.pallas import tpu_sc as plsc

N = 10000
E = 320000
D_IN = 128
H = 64
D_OUT = 128

NC = 2
NS = 16
NW = NC * NS

B = 128
NK = 80
NBUF = 2
E_PAD = NW * NK * B
STRIPE = 640
NP = NS * STRIPE
DUMMY = N

_f32 = jnp.float32


def _sc_body(y_hbm, src_hbm, dst_hbm, part0, part1, src_v, dst_v, rows, tbl,
             acc, gs0, gs1, gs2, ss0):
    c = lax.axis_index("c")
    s = lax.axis_index("s")
    wid = c * NS + s
    base = s * STRIPE

    d_src = pltpu.async_copy(src_hbm.at[wid], src_v, gs1)
    d_dst = pltpu.async_copy(dst_hbm.at[wid], dst_v, gs2)
    d_tbl = pltpu.async_copy(
        y_hbm.at[pl.ds(base, STRIPE)], tbl.at[pl.ds(base, STRIPE)], gs0)

    z = jnp.zeros((16,), _f32)

    def zrow(r, carry):
        for g in range(H // 16):
            rows[0, r, pl.ds(g * 16, 16)] = z
        return carry

    lax.fori_loop(0, B, zrow, 0)
    d_z = [
        pltpu.async_copy(rows.at[0], acc.at[pl.ds(base + j * B, B)], ss0)
        for j in range(STRIPE // B)
    ]
    for d in (d_src, d_dst, d_tbl, *d_z):
        d.wait()
    plsc.subcore_barrier()

    pltpu.async_copy(tbl.at[src_v.at[0]], rows.at[0], gs0)

    def chunk(k, carry):
        b = lax.rem(k, 2)
        nxt = jnp.minimum(k + 1, NK - 1)
        pltpu.make_async_copy(tbl.at[src_v.at[k]], rows.at[b], gs0).wait()
        pltpu.async_copy(tbl.at[src_v.at[nxt]], rows.at[1 - b], gs0)
        pltpu.sync_copy(rows.at[b], acc.at[dst_v.at[k]], add=True)
        return carry

    lax.fori_loop(0, NK, chunk, 0)
    pltpu.make_async_copy(tbl.at[src_v.at[0]], rows.at[NK % 2], gs0).wait()
    plsc.subcore_barrier()

    @pl.when(c == 0)
    def _():
        pltpu.sync_copy(acc.at[pl.ds(base, STRIPE)], part0.at[pl.ds(base, STRIPE)])

    @pl.when(c == 1)
    def _():
        pltpu.sync_copy(acc.at[pl.ds(base, STRIPE)], part1.at[pl.ds(base, STRIPE)])


_sc_segsum = functools.partial(
    pl.kernel,
    out_type=(
        jax.ShapeDtypeStruct((NP, H), _f32),
        jax.ShapeDtypeStruct((NP, H), _f32),
    ),
    mesh=plsc.VectorSubcoreMesh(core_axis_name="c", subcore_axis_name="s"),
    compiler_params=pltpu.CompilerParams(use_tc_tiling_on_sc=False),
    scratch_types=[
        pltpu.VMEM((NK, B), jnp.int32),
        pltpu.VMEM((NK, B), jnp.int32),
        pltpu.VMEM((NBUF, B, H), _f32),
        pltpu.VMEM_SHARED((NP, H), _f32),
        pltpu.VMEM_SHARED((NP, H), _f32),
    ] + [pltpu.SemaphoreType.DMA] * 4,
)(_sc_body)


_DN = (((1,), (1,)), ((), ()))


def _lrelu(v):
    return jnp.where(v >= 0, v, 0.01 * v)


def _t0_body(x_ref, wrel_ref, wroot_ref, b_ref, y_ref, r_ref):
    y_ref[...] = lax.dot_general(x_ref[...], wrel_ref[...], _DN,
                                 preferred_element_type=_f32)
    r_ref[...] = b_ref[...] + lax.dot_general(x_ref[...], wroot_ref[...], _DN,
                                              preferred_element_type=_f32)


def _comb1_body(p0_ref, p1_ref, r_ref, wrel_ref, wroot_ref, b_ref, y_ref,
                r2_ref):
    h = _lrelu(p0_ref[...] + p1_ref[...] + r_ref[...])
    y_ref[...] = lax.dot_general(h, wrel_ref[...], _DN,
                                 preferred_element_type=_f32)
    r2_ref[...] = b_ref[...] + lax.dot_general(h, wroot_ref[...], _DN,
                                               preferred_element_type=_f32)


def _comb2_body(p0_ref, p1_ref, r_ref, wroot_ref, b_ref, h_ref, r2_ref):
    h = _lrelu(p0_ref[...] + p1_ref[...] + r_ref[...])
    h_ref[...] = h
    r2_ref[...] = b_ref[...] + lax.dot_general(h, wroot_ref[...], _DN,
                                               preferred_element_type=_f32)


def _comb3_body(p0_ref, p1_ref, wrel_ref, r_ref, out_ref):
    agg = p0_ref[...] + p1_ref[...]
    rel = lax.dot_general(agg, wrel_ref[...], _DN,
                          preferred_element_type=_f32)
    out_ref[...] = _lrelu(rel + r_ref[...])


_GRID = (N // 1000,)


def _row_spec(w):
    return pl.BlockSpec((1000, w), lambda i: (i, 0))


def _full_spec(shape):
    return pl.BlockSpec(shape, lambda i: (0,) * len(shape))


def _tc_t0(x, wrel, wroot, b):
    return pl.pallas_call(
        _t0_body,
        grid=_GRID,
        in_specs=[_row_spec(D_IN), _full_spec(wrel.shape),
                  _full_spec(wroot.shape), _full_spec((1, H))],
        out_specs=[_row_spec(H), _row_spec(H)],
        out_shape=[jax.ShapeDtypeStruct((NP, H), _f32),
                   jax.ShapeDtypeStruct((NP, H), _f32)],
    )(x, wrel, wroot, b.reshape(1, H))


def _tc_comb1(p0, p1, r, wrel, wroot, b):
    return pl.pallas_call(
        _comb1_body,
        grid=_GRID,
        in_specs=[_row_spec(H), _row_spec(H), _row_spec(H),
                  _full_spec(wrel.shape), _full_spec(wroot.shape),
                  _full_spec((1, H))],
        out_specs=[_row_spec(H), _row_spec(H)],
        out_shape=[jax.ShapeDtypeStruct((NP, H), _f32),
                   jax.ShapeDtypeStruct((NP, H), _f32)],
    )(p0, p1, r, wrel, wroot, b.reshape(1, H))


def _tc_comb2(p0, p1, r, wroot, b):
    return pl.pallas_call(
        _comb2_body,
        grid=_GRID,
        in_specs=[_row_spec(H), _row_spec(H), _row_spec(H),
                  _full_spec(wroot.shape), _full_spec((1, D_OUT))],
        out_specs=[_row_spec(H), _row_spec(D_OUT)],
        out_shape=[jax.ShapeDtypeStruct((NP, H), _f32),
                   jax.ShapeDtypeStruct((NP, D_OUT), _f32)],
    )(p0, p1, r, wroot, b.reshape(1, D_OUT))


def _tc_comb3(p0, p1, wrel, r):
    return pl.pallas_call(
        _comb3_body,
        grid=_GRID,
        in_specs=[_row_spec(H), _row_spec(H), _full_spec(wrel.shape),
                  _row_spec(D_OUT)],
        out_specs=_row_spec(D_OUT),
        out_shape=jax.ShapeDtypeStruct((N, D_OUT), _f32),
    )(p0, p1, wrel, r)


def kernel(x, edge_index, batch, Wrel0, brel0, Wroot0, Wrel1, brel1, Wroot1,
           Wrel2, brel2, Wroot2):
    pad = E_PAD - E
    src = jnp.concatenate([edge_index[0], jnp.zeros((pad,), jnp.int32)])
    dummy = DUMMY + jax.lax.rem(jnp.arange(pad, dtype=jnp.int32), NP - N)
    dst = jnp.concatenate([edge_index[1], dummy])
    src3 = src.reshape(NW, NK, B)
    dst3 = dst.reshape(NW, NK, B)

    y0, r0 = _tc_t0(x, Wrel0, Wroot0, brel0)
    a0, b0 = _sc_segsum(y0, src3, dst3)
    y1, r1 = _tc_comb1(a0, b0, r0, Wrel1, Wroot1, brel1)
    a1, b1 = _sc_segsum(y1, src3, dst3)
    h2, r2 = _tc_comb2(a1, b1, r1, Wroot2, brel2)
    a2, b2 = _sc_segsum(h2, src3, dst3)
    return _tc_comb3(a2, b2, Wrel2, r2)
